# baseline (speedup 1.0000x reference)
import functools

import jax
import jax.numpy as jnp
from jax.experimental import pallas as pl
from jax.experimental.pallas import tpu as pltpu

N = 2048
E = 65536
D = 256
H = 8
HD = D // H
EB = 2048
NB = E // EB

_F32 = jnp.float32
_BF16 = jnp.bfloat16


def _bn(x, w, b, eps=1e-5):
    mean = jnp.mean(x, axis=0, keepdims=True)
    xc = x - mean
    var = jnp.mean(xc * xc, axis=0, keepdims=True)
    return xc / jnp.sqrt(var + eps) * w + b


def _node_mlp(x, n1w, n1b, n2w, n2b):
    xh = jax.nn.silu(jnp.dot(x, n1w, preferred_element_type=_F32) + n1b)
    return jnp.dot(xh, n2w, preferred_element_type=_F32) + n2b


def _sph_rowsT(rowoh, colohT, pos8_ref, posT8_ref):
    pr = jnp.dot(rowoh, pos8_ref[...], preferred_element_type=_F32)
    prT = jax.lax.transpose(pr, (1, 0))
    pcT = jnp.dot(posT8_ref[...], colohT, preferred_element_type=_F32)
    relT = prT - pcT
    rx, ry, rz = relT[0:1, :], relT[1:2, :], relT[2:3, :]
    s = rx * rx + ry * ry + rz * rz
    el = jnp.sqrt(s + 1e-12)
    scale = 1.0 / (jnp.sqrt(s) + 1e-10 * el)
    dx, dy, dz = rx * scale, ry * scale, rz * scale
    return jnp.concatenate([
        jnp.full_like(dx, 0.28209479177387814),
        0.4886025119029199 * dx, 0.4886025119029199 * dy,
        0.4886025119029199 * dz,
        1.0925484305920792 * dx * dy,
        1.0925484305920792 * dy * dz,
        0.31539156525252005 * (3.0 * dz * dz - 1.0),
        1.0925484305920792 * dx * dz,
        0.5462742152960396 * (dx * dx - dy * dy),
        jnp.zeros((7, dx.shape[1]), _F32)], axis=0)


def _edge_mlp(sphT, e1w_ref, e1b_ref, e2w_ref, e2b_ref):
    h1 = jax.lax.dot_general(sphT, e1w_ref[...], (((0,), (0,)), ((), ())),
                             preferred_element_type=_F32) + e1b_ref[...]
    h1 = jax.nn.silu(h1).astype(_BF16)
    return (jnp.dot(h1, e2w_ref[...], preferred_element_type=_F32)
            + e2b_ref[...])


def _conv1_kernel(row_ref, colt_ref, x_ref, pos8_ref, posT8_ref,
                  n1w_ref, n1b_ref, n2w_ref, n2b_ref,
                  e1w1_ref, e1b1_ref, e2w1_ref, e2b1_ref,
                  e1w2_ref, e1b2_ref, e2w2_ref, e2b2_ref,
                  oaw_ref, oxw_ref, o1b_ref, o2w_ref, o2b_ref,
                  bnw_ref, bnb_ref, nw_ref, nb_ref,
                  m1w_ref, m1b_ref, m2w_ref, m2b_ref,
                  h_ref, xt2_ref, e2_ref, cnt_ref,
                  xtb_s, agg_s, cnt_s):
    i = pl.program_id(0)

    @pl.when(i == 0)
    def _init():
        xtb_s[...] = _node_mlp(x_ref[...], n1w_ref[...], n1b_ref[...],
                               n2w_ref[...], n2b_ref[...]).astype(_BF16)
        agg_s[...] = jnp.zeros_like(agg_s)
        cnt_s[...] = jnp.zeros_like(cnt_s)

    row = row_ref[...]
    colt = colt_ref[0]
    iota = jax.lax.broadcasted_iota(jnp.int32, (EB, N), 1)
    iota_t = jax.lax.broadcasted_iota(jnp.int32, (N, EB), 0)
    rowoh = (row == iota).astype(_BF16)
    colohT = (colt == iota_t).astype(_BF16)
    cnt_s[...] += jnp.sum(colohT.astype(_F32), axis=1, keepdims=True)

    sphT = _sph_rowsT(rowoh, colohT, pos8_ref, posT8_ref)
    e1 = _edge_mlp(sphT, e1w1_ref, e1b1_ref, e2w1_ref, e2b1_ref)
    e2_ref[...] = _edge_mlp(sphT, e1w2_ref, e1b2_ref, e2w2_ref, e2b2_ref)

    xtg = jnp.dot(rowoh, xtb_s[...], preferred_element_type=_F32)
    msgs = (xtg * e1).astype(_BF16)
    agg_s[...] += jnp.dot(colohT, msgs, preferred_element_type=_F32)

    @pl.when(i == NB - 1)
    def _finalize():
        cnt_ref[...] = cnt_s[...]
        cnt = jnp.maximum(cnt_s[...], 1.0)
        agg = agg_s[...] / cnt
        g1 = jax.nn.silu(
            jnp.dot(agg, oaw_ref[...], preferred_element_type=_F32)
            + jnp.dot(x_ref[...], oxw_ref[...], preferred_element_type=_F32)
            + o1b_ref[...])
        out = (jnp.dot(g1, o2w_ref[...], preferred_element_type=_F32)
               + o2b_ref[...])
        out = _bn(out, bnw_ref[...], bnb_ref[...])
        h = jax.nn.silu(_bn(out, nw_ref[...], nb_ref[...]))
        h_ref[...] = h
        xt2_ref[...] = _node_mlp(h, m1w_ref[...], m1b_ref[...],
                                 m2w_ref[...], m2b_ref[...])


def _conv_kernel(with_xt2, row_ref, colt_ref, e_ref, xin_ref, xt_ref,
                 cnt_ref,
                 oaw_ref, oxw_ref, o1b_ref, o2w_ref, o2b_ref,
                 bnw_ref, bnb_ref, nw_ref, nb_ref,
                 *rest):
    if with_xt2:
        (n1w_ref, n1b_ref, n2w_ref, n2b_ref,
         h_ref, xt2_ref, xtb_s, agg_s) = rest
    else:
        h_ref, xtb_s, agg_s = rest
    i = pl.program_id(0)

    @pl.when(i == 0)
    def _init():
        xtb_s[...] = xt_ref[...].astype(_BF16)
        agg_s[...] = jnp.zeros_like(agg_s)

    row = row_ref[...]
    colt = colt_ref[0]
    iota = jax.lax.broadcasted_iota(jnp.int32, (EB, N), 1)
    iota_t = jax.lax.broadcasted_iota(jnp.int32, (N, EB), 0)
    rowoh = (row == iota).astype(_BF16)
    colohT = (colt == iota_t).astype(_BF16)

    xtg = jnp.dot(rowoh, xtb_s[...], preferred_element_type=_F32)
    msgs = (xtg * e_ref[...]).astype(_BF16)
    agg_s[...] += jnp.dot(colohT, msgs, preferred_element_type=_F32)

    @pl.when(i == NB - 1)
    def _finalize():
        cnt = jnp.maximum(cnt_ref[...], 1.0)
        agg = agg_s[...] / cnt
        g1 = jax.nn.silu(
            jnp.dot(agg, oaw_ref[...], preferred_element_type=_F32)
            + jnp.dot(xin_ref[...], oxw_ref[...], preferred_element_type=_F32)
            + o1b_ref[...])
        out = (jnp.dot(g1, o2w_ref[...], preferred_element_type=_F32)
               + o2b_ref[...])
        out = _bn(out, bnw_ref[...], bnb_ref[...])
        h = jax.nn.silu(_bn(out, nw_ref[...], nb_ref[...]))
        h_ref[...] = h
        if with_xt2:
            xt2_ref[...] = _node_mlp(h, n1w_ref[...], n1b_ref[...],
                                     n2w_ref[...], n2b_ref[...])


def _attn_kernel(h_ref, wq_ref, bq_ref, wk_ref, bk_ref, wv_ref, bv_ref,
                 wo_ref, bo_ref, out_ref):
    i = pl.program_id(0)
    h = h_ref[...].astype(_BF16)
    q = jnp.dot(h, wq_ref[0], preferred_element_type=_F32) + bq_ref[0]
    k = jnp.dot(h, wk_ref[0], preferred_element_type=_F32) + bk_ref[0]
    v = jnp.dot(h, wv_ref[0], preferred_element_type=_F32) + bv_ref[0]
    s = jax.lax.dot_general(q.astype(_BF16), k.astype(_BF16),
                            (((1,), (1,)), ((), ())),
                            preferred_element_type=_F32) * (HD ** -0.5)
    p = jax.nn.softmax(s, axis=-1)
    o = jnp.dot(p.astype(_BF16), v.astype(_BF16),
                preferred_element_type=_F32)
    contrib = jnp.dot(o.astype(_BF16), wo_ref[0], preferred_element_type=_F32)

    @pl.when(i == 0)
    def _first():
        out_ref[...] = contrib + bo_ref[...]

    @pl.when(i > 0)
    def _rest():
        out_ref[...] += contrib


def _full(shape):
    return pl.BlockSpec(shape, lambda i: (0,) * len(shape))


def _eblk():
    return pl.BlockSpec((EB, 1), lambda i: (i, 0))


def _erow():
    return pl.BlockSpec((EB, D), lambda i: (i, 0))


def _conv1_call(row, colt, x, pos8, posT8, w):
    in_specs = [_eblk(), pl.BlockSpec((1, 1, EB), lambda i: (i, 0, 0)),
                _full((N, D)), _full((N, 8)), _full((8, N))]
    in_specs += [_full(a.shape) for a in w]
    return pl.pallas_call(
        _conv1_kernel,
        grid=(NB,),
        in_specs=in_specs,
        out_specs=[_full((N, D)), _full((N, D)), _erow(), _full((N, 1))],
        out_shape=[jax.ShapeDtypeStruct((N, D), _F32),
                   jax.ShapeDtypeStruct((N, D), _F32),
                   jax.ShapeDtypeStruct((E, D), _F32),
                   jax.ShapeDtypeStruct((N, 1), _F32)],
        scratch_shapes=[pltpu.VMEM((N, D), _BF16),
                        pltpu.VMEM((N, D), _F32),
                        pltpu.VMEM((N, 1), _F32)],
    )(row, colt, x, pos8, posT8, *w)


def _conv_call(row, colt, e_all, xin, xt, cnt_col, w, node_w):
    with_xt2 = node_w is not None
    allw = list(w) + (list(node_w) if with_xt2 else [])
    in_specs = ([_eblk(), pl.BlockSpec((1, 1, EB), lambda i: (i, 0, 0)),
                 _erow(), _full((N, D)), _full((N, D)),
                 _full((N, 1))] + [_full(a.shape) for a in allw])
    out_specs = [_full((N, D))] * (2 if with_xt2 else 1)
    out_shape = [jax.ShapeDtypeStruct((N, D), _F32)] * (2 if with_xt2 else 1)
    res = pl.pallas_call(
        functools.partial(_conv_kernel, with_xt2),
        grid=(NB,),
        in_specs=in_specs,
        out_specs=out_specs if with_xt2 else out_specs[0],
        out_shape=out_shape if with_xt2 else out_shape[0],
        scratch_shapes=[pltpu.VMEM((N, D), _BF16),
                        pltpu.VMEM((N, D), _F32)],
    )(row, colt, e_all, xin, xt, cnt_col, *allw)
    return res


def _attn_call(h, p):
    wq = p["attn_q_w"].reshape(D, H, HD).transpose(1, 0, 2).astype(_BF16)
    wk = p["attn_k_w"].reshape(D, H, HD).transpose(1, 0, 2).astype(_BF16)
    wv = p["attn_v_w"].reshape(D, H, HD).transpose(1, 0, 2).astype(_BF16)
    bq = p["attn_q_b"].reshape(H, 1, HD)
    bk = p["attn_k_b"].reshape(H, 1, HD)
    bv = p["attn_v_b"].reshape(H, 1, HD)
    wo = p["attn_o_w"].reshape(H, HD, D).astype(_BF16)
    bo = p["attn_o_b"].reshape(1, D)
    hw = pl.BlockSpec((1, D, HD), lambda i: (i, 0, 0))
    hb = pl.BlockSpec((1, 1, HD), lambda i: (i, 0, 0))
    ho = pl.BlockSpec((1, HD, D), lambda i: (i, 0, 0))
    return pl.pallas_call(
        _attn_kernel,
        grid=(H,),
        in_specs=[_full((N, D)), hw, hb, hw, hb, hw, hb, ho, _full((1, D))],
        out_specs=_full((N, D)),
        out_shape=jax.ShapeDtypeStruct((N, D), _F32),
    )(h, wq, bq, wk, bk, wv, bv, wo, bo)


def _pad16(w):
    return jnp.pad(w, ((0, 16 - w.shape[0]), (0, 0)))


def kernel(x, edge_index, edge_attr, pos, params):
    p = params
    b = lambda name: p[name + "_b"].reshape(1, D)
    row = edge_index[0].reshape(E, 1)
    colt = edge_index[1].reshape(NB, 1, EB)
    pos8 = jnp.pad(pos, ((0, 0), (0, 8 - pos.shape[1]))).astype(_BF16)
    posT8 = pos8.T

    def conv_w(prefix):
        o1w = p[prefix + "_out1_w"]
        return (o1w[:D], o1w[D:], b(prefix + "_out1"),
                p[prefix + "_out2_w"], b(prefix + "_out2"),
                p[prefix + "_bn_w"].reshape(1, D), p[prefix + "_bn_b"].reshape(1, D))

    bn1 = (p["norm1_w"].reshape(1, D), p["norm1_b"].reshape(1, D))
    bn2 = (p["norm2_w"].reshape(1, D), p["norm2_b"].reshape(1, D))
    node2_w = (p["conv2_node1_w"], b("conv2_node1"),
               p["conv2_node2_w"], b("conv2_node2"))

    c1_w = ((p["conv1_node1_w"], b("conv1_node1"),
             p["conv1_node2_w"], b("conv1_node2"),
             _pad16(p["conv1_edge1_w"]), b("conv1_edge1"),
             p["conv1_edge2_w"].astype(_BF16), b("conv1_edge2"),
             _pad16(p["conv2_edge1_w"]), b("conv2_edge1"),
             p["conv2_edge2_w"].astype(_BF16), b("conv2_edge2"))
            + conv_w("conv1") + bn1 + node2_w)
    h1, xt2, e2_all, cnt_col = _conv1_call(row, colt, x, pos8, posT8, c1_w)
    h2 = _conv_call(row, colt, e2_all, h1, xt2, cnt_col,
                    conv_w("conv2") + bn2, None)
    return _attn_call(h2, p)


# speedup vs baseline: 1.0160x; 1.0160x over previous
import functools

import jax
import jax.numpy as jnp
from jax.experimental import pallas as pl
from jax.experimental.pallas import tpu as pltpu

N = 2048
E = 65536
D = 256
H = 8
HD = D // H
EB = 2048
NB = E // EB

_F32 = jnp.float32
_BF16 = jnp.bfloat16


def _bn(x, w, b, eps=1e-5):
    mean = jnp.mean(x, axis=0, keepdims=True)
    xc = x - mean
    var = jnp.mean(xc * xc, axis=0, keepdims=True)
    return xc / jnp.sqrt(var + eps) * w + b


def _node_mlp(x, n1w, n1b, n2w, n2b):
    xh = jax.nn.silu(jnp.dot(x, n1w, preferred_element_type=_F32) + n1b)
    return jnp.dot(xh, n2w, preferred_element_type=_F32) + n2b


def _prep_kernel(row_ref, col_ref, x_ref, pos_ref,
                 n1w_ref, n1b_ref, n2w_ref, n2b_ref,
                 e1w1_ref, e1b1_ref, e2w1_ref, e2b1_ref,
                 e1w2_ref, e1b2_ref, e2w2_ref, e2b2_ref,
                 e1_ref, e2_ref, xt1_ref, cnt_ref, cnt_s):
    i = pl.program_id(0)

    @pl.when(i == 0)
    def _init():
        xt1_ref[...] = _node_mlp(x_ref[...], n1w_ref[...], n1b_ref[...],
                                 n2w_ref[...], n2b_ref[...])
        cnt_s[...] = jnp.zeros_like(cnt_s)

    row = row_ref[...]
    col = col_ref[...]
    iota = jax.lax.broadcasted_iota(jnp.int32, (EB, N), 1)
    rowoh = (row == iota).astype(_BF16)
    coloh = (col == iota).astype(_BF16)
    cnt_s[0:1, :] += jnp.sum(coloh.astype(_F32), axis=0, keepdims=True)

    rel = jnp.dot(rowoh - coloh, pos_ref[...],
                  preferred_element_type=_F32)
    relT = jax.lax.transpose(rel[:, 0:8], (1, 0))
    rx, ry, rz = relT[0:1, :], relT[1:2, :], relT[2:3, :]
    s = rx * rx + ry * ry + rz * rz
    el = jnp.sqrt(s + 1e-12)
    scale = 1.0 / (jnp.sqrt(s) + 1e-10 * el)
    dx, dy, dz = rx * scale, ry * scale, rz * scale

    sphT = jnp.concatenate([
        jnp.full_like(dx, 0.28209479177387814),
        0.4886025119029199 * dx, 0.4886025119029199 * dy,
        0.4886025119029199 * dz,
        1.0925484305920792 * dx * dy,
        1.0925484305920792 * dy * dz,
        0.31539156525252005 * (3.0 * dz * dz - 1.0),
        1.0925484305920792 * dx * dz,
        0.5462742152960396 * (dx * dx - dy * dy),
        jnp.zeros((7, EB), _F32)], axis=0)

    for e1w, e1b, e2w, e2b, e_ref in (
            (e1w1_ref, e1b1_ref, e2w1_ref, e2b1_ref, e1_ref),
            (e1w2_ref, e1b2_ref, e2w2_ref, e2b2_ref, e2_ref)):
        h1 = jax.lax.dot_general(sphT, e1w[...], (((0,), (0,)), ((), ())),
                                 preferred_element_type=_F32) + e1b[...]
        h1 = jax.nn.silu(h1).astype(_BF16)
        e_ref[...] = (jnp.dot(h1, e2w[...], preferred_element_type=_F32)
                      + e2b[...])

    @pl.when(i == NB - 1)
    def _fin():
        cnt_ref[...] = cnt_s[0:1, :]


def _conv_kernel(with_xt2, row_ref, colt_ref, e_ref, xin_ref, xt_ref,
                 cnt_ref,
                 oaw_ref, oxw_ref, o1b_ref, o2w_ref, o2b_ref,
                 bnw_ref, bnb_ref, nw_ref, nb_ref,
                 *rest):
    if with_xt2:
        (n1w_ref, n1b_ref, n2w_ref, n2b_ref,
         h_ref, xt2_ref, xtb_s, agg_s) = rest
    else:
        h_ref, xtb_s, agg_s = rest
    i = pl.program_id(0)

    @pl.when(i == 0)
    def _init():
        xtb_s[...] = xt_ref[...].astype(_BF16)
        agg_s[...] = jnp.zeros_like(agg_s)

    row = row_ref[...]
    colt = colt_ref[0]
    iota = jax.lax.broadcasted_iota(jnp.int32, (EB, N), 1)
    iota_t = jax.lax.broadcasted_iota(jnp.int32, (N, EB), 0)
    rowoh = (row == iota).astype(_BF16)
    colohT = (colt == iota_t).astype(_BF16)

    xtg = jnp.dot(rowoh, xtb_s[...], preferred_element_type=_F32)
    msgs = (xtg * e_ref[...]).astype(_BF16)
    agg_s[...] += jnp.dot(colohT, msgs, preferred_element_type=_F32)

    @pl.when(i == NB - 1)
    def _finalize():
        cnt = jnp.maximum(cnt_ref[...], 1.0)
        agg = agg_s[...] / cnt
        g1 = jax.nn.silu(
            jnp.dot(agg, oaw_ref[...], preferred_element_type=_F32)
            + jnp.dot(xin_ref[...], oxw_ref[...], preferred_element_type=_F32)
            + o1b_ref[...])
        out = (jnp.dot(g1, o2w_ref[...], preferred_element_type=_F32)
               + o2b_ref[...])
        out = _bn(out, bnw_ref[...], bnb_ref[...])
        h = jax.nn.silu(_bn(out, nw_ref[...], nb_ref[...]))
        h_ref[...] = h
        if with_xt2:
            xt2_ref[...] = _node_mlp(h, n1w_ref[...], n1b_ref[...],
                                     n2w_ref[...], n2b_ref[...])


def _attn_kernel(h_ref, wq_ref, bq_ref, wk_ref, bk_ref, wv_ref, bv_ref,
                 wo_ref, bo_ref, out_ref):
    i = pl.program_id(0)
    h = h_ref[...].astype(_BF16)
    q = jnp.dot(h, wq_ref[0], preferred_element_type=_F32) + bq_ref[0]
    k = jnp.dot(h, wk_ref[0], preferred_element_type=_F32) + bk_ref[0]
    v = jnp.dot(h, wv_ref[0], preferred_element_type=_F32) + bv_ref[0]
    s = jax.lax.dot_general(q.astype(_BF16), k.astype(_BF16),
                            (((1,), (1,)), ((), ())),
                            preferred_element_type=_F32) * (HD ** -0.5)
    p = jax.nn.softmax(s, axis=-1)
    o = jnp.dot(p.astype(_BF16), v.astype(_BF16),
                preferred_element_type=_F32)
    contrib = jnp.dot(o.astype(_BF16), wo_ref[0], preferred_element_type=_F32)

    @pl.when(i == 0)
    def _first():
        out_ref[...] = contrib + bo_ref[...]

    @pl.when(i > 0)
    def _rest():
        out_ref[...] += contrib


def _full(shape):
    return pl.BlockSpec(shape, lambda i: (0,) * len(shape))


def _eblk():
    return pl.BlockSpec((EB, 1), lambda i: (i, 0))


def _erow():
    return pl.BlockSpec((EB, D), lambda i: (i, 0))


def _prep_call(row, col, x, pos_p, w):
    in_specs = [_eblk(), _eblk(), _full((N, D)), _full((N, 128))]
    in_specs += [_full(a.shape) for a in w]
    return pl.pallas_call(
        _prep_kernel,
        grid=(NB,),
        in_specs=in_specs,
        out_specs=[_erow(), _erow(), _full((N, D)), _full((1, N))],
        out_shape=[jax.ShapeDtypeStruct((E, D), _F32),
                   jax.ShapeDtypeStruct((E, D), _F32),
                   jax.ShapeDtypeStruct((N, D), _F32),
                   jax.ShapeDtypeStruct((1, N), _F32)],
        scratch_shapes=[pltpu.VMEM((8, N), _F32)],
    )(row, col, x, pos_p, *w)


def _conv_call(row, colt, e_all, xin, xt, cnt_col, w, node_w):
    with_xt2 = node_w is not None
    allw = list(w) + (list(node_w) if with_xt2 else [])
    in_specs = ([_eblk(), pl.BlockSpec((1, 1, EB), lambda i: (i, 0, 0)),
                 _erow(), _full((N, D)), _full((N, D)),
                 _full((N, 1))] + [_full(a.shape) for a in allw])
    out_specs = [_full((N, D))] * (2 if with_xt2 else 1)
    out_shape = [jax.ShapeDtypeStruct((N, D), _F32)] * (2 if with_xt2 else 1)
    res = pl.pallas_call(
        functools.partial(_conv_kernel, with_xt2),
        grid=(NB,),
        in_specs=in_specs,
        out_specs=out_specs if with_xt2 else out_specs[0],
        out_shape=out_shape if with_xt2 else out_shape[0],
        scratch_shapes=[pltpu.VMEM((N, D), _BF16),
                        pltpu.VMEM((N, D), _F32)],
    )(row, colt, e_all, xin, xt, cnt_col, *allw)
    return res


def _attn_call(h, p):
    wq = p["attn_q_w"].reshape(D, H, HD).transpose(1, 0, 2).astype(_BF16)
    wk = p["attn_k_w"].reshape(D, H, HD).transpose(1, 0, 2).astype(_BF16)
    wv = p["attn_v_w"].reshape(D, H, HD).transpose(1, 0, 2).astype(_BF16)
    bq = p["attn_q_b"].reshape(H, 1, HD)
    bk = p["attn_k_b"].reshape(H, 1, HD)
    bv = p["attn_v_b"].reshape(H, 1, HD)
    wo = p["attn_o_w"].reshape(H, HD, D).astype(_BF16)
    bo = p["attn_o_b"].reshape(1, D)
    hw = pl.BlockSpec((1, D, HD), lambda i: (i, 0, 0))
    hb = pl.BlockSpec((1, 1, HD), lambda i: (i, 0, 0))
    ho = pl.BlockSpec((1, HD, D), lambda i: (i, 0, 0))
    return pl.pallas_call(
        _attn_kernel,
        grid=(H,),
        in_specs=[_full((N, D)), hw, hb, hw, hb, hw, hb, ho, _full((1, D))],
        out_specs=_full((N, D)),
        out_shape=jax.ShapeDtypeStruct((N, D), _F32),
    )(h, wq, bq, wk, bk, wv, bv, wo, bo)


def _pad16(w):
    return jnp.pad(w, ((0, 16 - w.shape[0]), (0, 0)))


def kernel(x, edge_index, edge_attr, pos, params):
    p = params
    b = lambda name: p[name + "_b"].reshape(1, D)
    row = edge_index[0].reshape(E, 1)
    col = edge_index[1].reshape(E, 1)
    pos_p = jnp.pad(pos, ((0, 0), (0, 128 - pos.shape[1]))).astype(_BF16)

    prep_w = (p["conv1_node1_w"], b("conv1_node1"),
              p["conv1_node2_w"], b("conv1_node2"),
              _pad16(p["conv1_edge1_w"]), b("conv1_edge1"),
              p["conv1_edge2_w"].astype(_BF16), b("conv1_edge2"),
              _pad16(p["conv2_edge1_w"]), b("conv2_edge1"),
              p["conv2_edge2_w"].astype(_BF16), b("conv2_edge2"))
    e1_all, e2_all, xt1, cnt = _prep_call(row, col, x, pos_p, prep_w)
    cnt_col = cnt.reshape(N, 1)

    def conv_w(prefix):
        o1w = p[prefix + "_out1_w"]
        return (o1w[:D], o1w[D:], b(prefix + "_out1"),
                p[prefix + "_out2_w"], b(prefix + "_out2"),
                p[prefix + "_bn_w"].reshape(1, D), p[prefix + "_bn_b"].reshape(1, D))

    bn1 = (p["norm1_w"].reshape(1, D), p["norm1_b"].reshape(1, D))
    bn2 = (p["norm2_w"].reshape(1, D), p["norm2_b"].reshape(1, D))
    node2_w = (p["conv2_node1_w"], b("conv2_node1"),
               p["conv2_node2_w"], b("conv2_node2"))

    colt = edge_index[1].reshape(NB, 1, EB)
    h1, xt2 = _conv_call(row, colt, e1_all, x, xt1, cnt_col,
                         conv_w("conv1") + bn1, node2_w)
    h2 = _conv_call(row, colt, e2_all, h1, xt2, cnt_col,
                    conv_w("conv2") + bn2, None)
    return _attn_call(h2, p)


# speedup vs baseline: 1.0174x; 1.0013x over previous
import functools

import jax
import jax.numpy as jnp
from jax.experimental import pallas as pl
from jax.experimental.pallas import tpu as pltpu

N = 2048
E = 65536
D = 256
H = 8
HD = D // H
EB = 2048
NB = E // EB

_F32 = jnp.float32
_BF16 = jnp.bfloat16


def _bn(x, w, b, eps=1e-5):
    mean = jnp.mean(x, axis=0, keepdims=True)
    xc = x - mean
    var = jnp.mean(xc * xc, axis=0, keepdims=True)
    return xc / jnp.sqrt(var + eps) * w + b


def _node_mlp(x, n1w, n1b, n2w, n2b):
    xh = jax.nn.silu(jnp.dot(x, n1w, preferred_element_type=_F32) + n1b)
    return jnp.dot(xh, n2w, preferred_element_type=_F32) + n2b


def _prep_kernel(row_ref, col_ref, x_ref, pos_ref,
                 n1w_ref, n1b_ref, n2w_ref, n2b_ref,
                 e1w1_ref, e1b1_ref, e2w1_ref, e2b1_ref,
                 e1w2_ref, e1b2_ref, e2w2_ref, e2b2_ref,
                 e1_ref, e2_ref, xt1_ref, cnt_ref, cnt_s):
    i = pl.program_id(0)

    @pl.when(i == 0)
    def _init():
        xt1_ref[...] = _node_mlp(x_ref[...], n1w_ref[...], n1b_ref[...],
                                 n2w_ref[...], n2b_ref[...])
        cnt_s[...] = jnp.zeros_like(cnt_s)

    row = row_ref[...]
    col = col_ref[...]
    iota = jax.lax.broadcasted_iota(jnp.int32, (EB, N), 1)
    rowoh = (row == iota).astype(_BF16)
    coloh = (col == iota).astype(_BF16)
    cnt_s[0:1, :] += jnp.sum(coloh.astype(_F32), axis=0, keepdims=True)

    rel = jnp.dot(rowoh - coloh, pos_ref[...],
                  preferred_element_type=_F32)
    relT = jax.lax.transpose(rel[:, 0:8], (1, 0))
    rx, ry, rz = relT[0:1, :], relT[1:2, :], relT[2:3, :]
    s = rx * rx + ry * ry + rz * rz
    el = jnp.sqrt(s + 1e-12)
    scale = 1.0 / (jnp.sqrt(s) + 1e-10 * el)
    dx, dy, dz = rx * scale, ry * scale, rz * scale

    sphT = jnp.concatenate([
        jnp.full_like(dx, 0.28209479177387814),
        0.4886025119029199 * dx, 0.4886025119029199 * dy,
        0.4886025119029199 * dz,
        1.0925484305920792 * dx * dy,
        1.0925484305920792 * dy * dz,
        0.31539156525252005 * (3.0 * dz * dz - 1.0),
        1.0925484305920792 * dx * dz,
        0.5462742152960396 * (dx * dx - dy * dy),
        jnp.zeros((7, EB), _F32)], axis=0)

    for e1w, e1b, e2w, e2b, e_ref in (
            (e1w1_ref, e1b1_ref, e2w1_ref, e2b1_ref, e1_ref),
            (e1w2_ref, e1b2_ref, e2w2_ref, e2b2_ref, e2_ref)):
        h1 = jax.lax.dot_general(sphT, e1w[...], (((0,), (0,)), ((), ())),
                                 preferred_element_type=_F32) + e1b[...]
        h1 = jax.nn.silu(h1).astype(_BF16)
        e_ref[...] = (jnp.dot(h1, e2w[...], preferred_element_type=_F32)
                      + e2b[...]).astype(_BF16)

    @pl.when(i == NB - 1)
    def _fin():
        cnt_ref[...] = cnt_s[0:1, :]


def _conv_kernel(with_xt2, row_ref, colt_ref, e_ref, xin_ref, xt_ref,
                 cnt_ref,
                 oaw_ref, oxw_ref, o1b_ref, o2w_ref, o2b_ref,
                 bnw_ref, bnb_ref, nw_ref, nb_ref,
                 *rest):
    if with_xt2:
        (n1w_ref, n1b_ref, n2w_ref, n2b_ref,
         h_ref, xt2_ref, xtb_s, agg_s) = rest
    else:
        h_ref, xtb_s, agg_s = rest
    i = pl.program_id(0)

    @pl.when(i == 0)
    def _init():
        xtb_s[...] = xt_ref[...].astype(_BF16)
        agg_s[...] = jnp.zeros_like(agg_s)

    row = row_ref[...]
    colt = colt_ref[0]
    iota = jax.lax.broadcasted_iota(jnp.int32, (EB, N), 1)
    iota_t = jax.lax.broadcasted_iota(jnp.int32, (N, EB), 0)
    rowoh = (row == iota).astype(_BF16)
    colohT = (colt == iota_t).astype(_BF16)

    xtg = jnp.dot(rowoh, xtb_s[...], preferred_element_type=_F32)
    msgs = (xtg * e_ref[...].astype(_F32)).astype(_BF16)
    agg_s[...] += jnp.dot(colohT, msgs, preferred_element_type=_F32)

    @pl.when(i == NB - 1)
    def _finalize():
        cnt = jnp.maximum(cnt_ref[...], 1.0)
        agg = agg_s[...] / cnt
        g1 = jax.nn.silu(
            jnp.dot(agg, oaw_ref[...], preferred_element_type=_F32)
            + jnp.dot(xin_ref[...], oxw_ref[...], preferred_element_type=_F32)
            + o1b_ref[...])
        out = (jnp.dot(g1, o2w_ref[...], preferred_element_type=_F32)
               + o2b_ref[...])
        out = _bn(out, bnw_ref[...], bnb_ref[...])
        h = jax.nn.silu(_bn(out, nw_ref[...], nb_ref[...]))
        h_ref[...] = h
        if with_xt2:
            xt2_ref[...] = _node_mlp(h, n1w_ref[...], n1b_ref[...],
                                     n2w_ref[...], n2b_ref[...])


def _attn_kernel(h_ref, wq_ref, bq_ref, wk_ref, bk_ref, wv_ref, bv_ref,
                 wo_ref, bo_ref, out_ref):
    i = pl.program_id(0)
    h = h_ref[...].astype(_BF16)
    q = jnp.dot(h, wq_ref[0], preferred_element_type=_F32) + bq_ref[0]
    k = jnp.dot(h, wk_ref[0], preferred_element_type=_F32) + bk_ref[0]
    v = jnp.dot(h, wv_ref[0], preferred_element_type=_F32) + bv_ref[0]
    s = jax.lax.dot_general(q.astype(_BF16), k.astype(_BF16),
                            (((1,), (1,)), ((), ())),
                            preferred_element_type=_F32) * (HD ** -0.5)
    p = jax.nn.softmax(s, axis=-1)
    o = jnp.dot(p.astype(_BF16), v.astype(_BF16),
                preferred_element_type=_F32)
    contrib = jnp.dot(o.astype(_BF16), wo_ref[0], preferred_element_type=_F32)

    @pl.when(i == 0)
    def _first():
        out_ref[...] = contrib + bo_ref[...]

    @pl.when(i > 0)
    def _rest():
        out_ref[...] += contrib


def _full(shape):
    return pl.BlockSpec(shape, lambda i: (0,) * len(shape))


def _eblk():
    return pl.BlockSpec((EB, 1), lambda i: (i, 0))


def _erow():
    return pl.BlockSpec((EB, D), lambda i: (i, 0))


def _prep_call(row, col, x, pos_p, w):
    in_specs = [_eblk(), _eblk(), _full((N, D)), _full((N, 128))]
    in_specs += [_full(a.shape) for a in w]
    return pl.pallas_call(
        _prep_kernel,
        grid=(NB,),
        in_specs=in_specs,
        out_specs=[_erow(), _erow(), _full((N, D)), _full((1, N))],
        out_shape=[jax.ShapeDtypeStruct((E, D), _BF16),
                   jax.ShapeDtypeStruct((E, D), _BF16),
                   jax.ShapeDtypeStruct((N, D), _F32),
                   jax.ShapeDtypeStruct((1, N), _F32)],
        scratch_shapes=[pltpu.VMEM((8, N), _F32)],
    )(row, col, x, pos_p, *w)


def _conv_call(row, colt, e_all, xin, xt, cnt_col, w, node_w):
    with_xt2 = node_w is not None
    allw = list(w) + (list(node_w) if with_xt2 else [])
    in_specs = ([_eblk(), pl.BlockSpec((1, 1, EB), lambda i: (i, 0, 0)),
                 _erow(), _full((N, D)), _full((N, D)),
                 _full((N, 1))] + [_full(a.shape) for a in allw])
    out_specs = [_full((N, D))] * (2 if with_xt2 else 1)
    out_shape = [jax.ShapeDtypeStruct((N, D), _F32)] * (2 if with_xt2 else 1)
    res = pl.pallas_call(
        functools.partial(_conv_kernel, with_xt2),
        grid=(NB,),
        in_specs=in_specs,
        out_specs=out_specs if with_xt2 else out_specs[0],
        out_shape=out_shape if with_xt2 else out_shape[0],
        scratch_shapes=[pltpu.VMEM((N, D), _BF16),
                        pltpu.VMEM((N, D), _F32)],
    )(row, colt, e_all, xin, xt, cnt_col, *allw)
    return res


def _attn_call(h, p):
    wq = p["attn_q_w"].reshape(D, H, HD).transpose(1, 0, 2).astype(_BF16)
    wk = p["attn_k_w"].reshape(D, H, HD).transpose(1, 0, 2).astype(_BF16)
    wv = p["attn_v_w"].reshape(D, H, HD).transpose(1, 0, 2).astype(_BF16)
    bq = p["attn_q_b"].reshape(H, 1, HD)
    bk = p["attn_k_b"].reshape(H, 1, HD)
    bv = p["attn_v_b"].reshape(H, 1, HD)
    wo = p["attn_o_w"].reshape(H, HD, D).astype(_BF16)
    bo = p["attn_o_b"].reshape(1, D)
    hw = pl.BlockSpec((1, D, HD), lambda i: (i, 0, 0))
    hb = pl.BlockSpec((1, 1, HD), lambda i: (i, 0, 0))
    ho = pl.BlockSpec((1, HD, D), lambda i: (i, 0, 0))
    return pl.pallas_call(
        _attn_kernel,
        grid=(H,),
        in_specs=[_full((N, D)), hw, hb, hw, hb, hw, hb, ho, _full((1, D))],
        out_specs=_full((N, D)),
        out_shape=jax.ShapeDtypeStruct((N, D), _F32),
    )(h, wq, bq, wk, bk, wv, bv, wo, bo)


def _pad16(w):
    return jnp.pad(w, ((0, 16 - w.shape[0]), (0, 0)))


def kernel(x, edge_index, edge_attr, pos, params):
    p = params
    b = lambda name: p[name + "_b"].reshape(1, D)
    row = edge_index[0].reshape(E, 1)
    col = edge_index[1].reshape(E, 1)
    pos_p = jnp.pad(pos, ((0, 0), (0, 128 - pos.shape[1]))).astype(_BF16)

    prep_w = (p["conv1_node1_w"], b("conv1_node1"),
              p["conv1_node2_w"], b("conv1_node2"),
              _pad16(p["conv1_edge1_w"]), b("conv1_edge1"),
              p["conv1_edge2_w"].astype(_BF16), b("conv1_edge2"),
              _pad16(p["conv2_edge1_w"]), b("conv2_edge1"),
              p["conv2_edge2_w"].astype(_BF16), b("conv2_edge2"))
    e1_all, e2_all, xt1, cnt = _prep_call(row, col, x, pos_p, prep_w)
    cnt_col = cnt.reshape(N, 1)

    def conv_w(prefix):
        o1w = p[prefix + "_out1_w"]
        return (o1w[:D], o1w[D:], b(prefix + "_out1"),
                p[prefix + "_out2_w"], b(prefix + "_out2"),
                p[prefix + "_bn_w"].reshape(1, D), p[prefix + "_bn_b"].reshape(1, D))

    bn1 = (p["norm1_w"].reshape(1, D), p["norm1_b"].reshape(1, D))
    bn2 = (p["norm2_w"].reshape(1, D), p["norm2_b"].reshape(1, D))
    node2_w = (p["conv2_node1_w"], b("conv2_node1"),
               p["conv2_node2_w"], b("conv2_node2"))

    colt = edge_index[1].reshape(NB, 1, EB)
    h1, xt2 = _conv_call(row, colt, e1_all, x, xt1, cnt_col,
                         conv_w("conv1") + bn1, node2_w)
    h2 = _conv_call(row, colt, e2_all, h1, xt2, cnt_col,
                    conv_w("conv2") + bn2, None)
    return _attn_call(h2, p)
